# baseline (speedup 1.0000x reference)
import functools

import jax
import jax.numpy as jnp
from jax import lax
from jax.experimental import pallas as pl
from jax.experimental.pallas import tpu as pltpu
from jax.experimental.pallas import tpu_sc as plsc

VOCAB = 1000000
MAX_SEQ_LEN = 2048
DIM = 64
BATCH = 1024
SEQ = 200

ROWS = BATCH * SEQ
CHUNK = SEQ
HALF = CHUNK // 2
NCHUNK = ROWS // CHUNK
NWORKERS = 32
CPW = NCHUNK // NWORKERS


def _positional_encoding(max_len, dim):
    pos = jnp.arange(max_len, dtype=jnp.float32)[:, None]
    div = jnp.exp(jnp.arange(0, dim, 2, dtype=jnp.float32) * (-jnp.log(10000.0) / dim))
    pe = jnp.zeros((max_len, dim), dtype=jnp.float32)
    pe = pe.at[:, 0::2].set(jnp.sin(pos * div))
    pe = pe.at[:, 1::2].set(jnp.cos(pos * div))
    return pe


def _sc_body(idx_hbm, tbl_hbm, pe_hbm, out_hbm, idx_v, rows_v, pe_v, gsem):
    wid = lax.axis_index("s") * 2 + lax.axis_index("c")

    pltpu.sync_copy(pe_hbm, pe_v)

    def chunk_body(i, carry):
        c = wid * CPW + i
        pltpu.sync_copy(idx_hbm.at[c], idx_v)
        cp0 = pltpu.async_copy(tbl_hbm.at[idx_v.at[0]], rows_v.at[pl.ds(0, HALF)], gsem)
        cp1 = pltpu.async_copy(tbl_hbm.at[idx_v.at[1]], rows_v.at[pl.ds(HALF, HALF)], gsem)
        cp0.wait()
        cp1.wait()

        def add_body(r, carry2):
            for k in range(DIM // 16):
                sl = pl.ds(k * 16, 16)
                rows_v[r, sl] = rows_v[r, sl] + pe_v[r, sl]
            return carry2

        lax.fori_loop(0, CHUNK, add_body, 0, unroll=2)
        pltpu.sync_copy(rows_v, out_hbm.at[pl.ds(c * CHUNK, CHUNK)])
        return carry

    lax.fori_loop(0, CPW, chunk_body, 0)


def kernel(X, token_table):
    idx = X.astype(jnp.int32).reshape(NCHUNK, 2, HALF)
    pe = _positional_encoding(SEQ, DIM)

    mesh = plsc.VectorSubcoreMesh(core_axis_name="c", subcore_axis_name="s")
    run = functools.partial(
        pl.kernel,
        mesh=mesh,
        compiler_params=pltpu.CompilerParams(use_tc_tiling_on_sc=False),
        out_type=jax.ShapeDtypeStruct((ROWS, DIM), jnp.float32),
        scratch_types=[
            pltpu.VMEM((2, HALF), jnp.int32),
            pltpu.VMEM((CHUNK, DIM), jnp.float32),
            pltpu.VMEM((SEQ, DIM), jnp.float32),
            pltpu.SemaphoreType.DMA,
        ],
    )(_sc_body)
    out = run(idx, token_table, pe)
    return out.reshape(BATCH, SEQ, DIM)


# speedup vs baseline: 1.1466x; 1.1466x over previous
import functools

import jax
import jax.numpy as jnp
from jax import lax
from jax.experimental import pallas as pl
from jax.experimental.pallas import tpu as pltpu
from jax.experimental.pallas import tpu_sc as plsc

VOCAB = 1000000
MAX_SEQ_LEN = 2048
DIM = 64
BATCH = 1024
SEQ = 200

ROWS = BATCH * SEQ
CHUNK = SEQ
HALF = CHUNK // 2
NCHUNK = ROWS // CHUNK
NWORKERS = 32
CPW = NCHUNK // NWORKERS


def _positional_encoding(max_len, dim):
    pos = jnp.arange(max_len, dtype=jnp.float32)[:, None]
    div = jnp.exp(jnp.arange(0, dim, 2, dtype=jnp.float32) * (-jnp.log(10000.0) / dim))
    pe = jnp.zeros((max_len, dim), dtype=jnp.float32)
    pe = pe.at[:, 0::2].set(jnp.sin(pos * div))
    pe = pe.at[:, 1::2].set(jnp.cos(pos * div))
    return pe


NBUF = 6
LOOKAHEAD = 3


def _sc_body(idx_hbm, tbl_hbm, pe_hbm, out_hbm, idx_v, rows_v, pe_v, gsem, wsem):
    wid = lax.axis_index("s") * 2 + lax.axis_index("c")

    pltpu.sync_copy(pe_hbm, pe_v)
    pltpu.sync_copy(idx_hbm.at[wid], idx_v)

    def start_gather(i):
        b = i % NBUF
        c0 = pltpu.async_copy(
            tbl_hbm.at[idx_v.at[2 * i]], rows_v.at[b, pl.ds(0, HALF)], gsem.at[b])
        c1 = pltpu.async_copy(
            tbl_hbm.at[idx_v.at[2 * i + 1]], rows_v.at[b, pl.ds(HALF, HALF)],
            gsem.at[b])
        return c0, c1

    def start_writeback(i):
        b = i % NBUF
        c = wid * CPW + i
        return pltpu.async_copy(
            rows_v.at[b], out_hbm.at[pl.ds(c * CHUNK, CHUNK)], wsem.at[b])

    gathers = {}
    wbs = {}
    for i in range(LOOKAHEAD):
        gathers[i] = start_gather(i)

    for i in range(CPW):
        b = i % NBUF
        g0, g1 = gathers.pop(i)
        g0.wait()
        g1.wait()

        def add_body(r, carry, _b=b):
            for k in range(DIM // 16):
                sl = pl.ds(k * 16, 16)
                rows_v[_b, r, sl] = rows_v[_b, r, sl] + pe_v[r, sl]
            return carry

        lax.fori_loop(0, CHUNK, add_body, 0, unroll=2)
        wbs[i] = start_writeback(i)

        j = i + LOOKAHEAD
        if j < CPW:
            if j >= NBUF:
                wbs.pop(j - NBUF).wait()
            gathers[j] = start_gather(j)

    for i in sorted(wbs):
        wbs[i].wait()


def kernel(X, token_table):
    idx = X.astype(jnp.int32).reshape(NWORKERS, CPW * 2, HALF)
    pe = _positional_encoding(SEQ, DIM)

    mesh = plsc.VectorSubcoreMesh(core_axis_name="c", subcore_axis_name="s")
    run = functools.partial(
        pl.kernel,
        mesh=mesh,
        compiler_params=pltpu.CompilerParams(use_tc_tiling_on_sc=False),
        out_type=jax.ShapeDtypeStruct((ROWS, DIM), jnp.float32),
        scratch_types=[
            pltpu.VMEM((CPW * 2, HALF), jnp.int32),
            pltpu.VMEM((NBUF, CHUNK, DIM), jnp.float32),
            pltpu.VMEM((SEQ, DIM), jnp.float32),
            pltpu.SemaphoreType.DMA((NBUF,)),
            pltpu.SemaphoreType.DMA((NBUF,)),
        ],
    )(_sc_body)
    out = run(idx, token_table, pe)
    return out.reshape(BATCH, SEQ, DIM)


# speedup vs baseline: 1.1500x; 1.0030x over previous
import functools

import jax
import jax.numpy as jnp
from jax import lax
from jax.experimental import pallas as pl
from jax.experimental.pallas import tpu as pltpu
from jax.experimental.pallas import tpu_sc as plsc

VOCAB = 1000000
DIM = 64
BATCH = 1024
SEQ = 200

SPLIT_A = 96
SPLIT_B = 104
NWORKERS = 32
CPW = BATCH // NWORKERS
NBUF = 6
LOOKAHEAD = 3


def _positional_encoding(max_len, dim):
    pos = jnp.arange(max_len, dtype=jnp.float32)[:, None]
    div = jnp.exp(jnp.arange(0, dim, 2, dtype=jnp.float32) * (-jnp.log(10000.0) / dim))
    pe = jnp.zeros((max_len, dim), dtype=jnp.float32)
    pe = pe.at[:, 0::2].set(jnp.sin(pos * div))
    pe = pe.at[:, 1::2].set(jnp.cos(pos * div))
    return pe


def _sc_body(idx_hbm, tbl_hbm, pe_hbm, out_hbm, idx_v, rows_v, pe_v, gsem, wsem):
    wid = lax.axis_index("s") * 2 + lax.axis_index("c")
    base = wid * CPW

    pltpu.sync_copy(pe_hbm, pe_v)
    pltpu.sync_copy(idx_hbm.at[pl.ds(base, CPW)], idx_v)

    def start_gather(i):
        b = i % NBUF
        c0 = pltpu.async_copy(
            tbl_hbm.at[idx_v.at[i, pl.ds(0, SPLIT_A)]],
            rows_v.at[b, pl.ds(0, SPLIT_A)], gsem.at[b])
        c1 = pltpu.async_copy(
            tbl_hbm.at[idx_v.at[i, pl.ds(SPLIT_A, SPLIT_B)]],
            rows_v.at[b, pl.ds(SPLIT_A, SPLIT_B)], gsem.at[b])
        return c0, c1

    def start_writeback(i):
        b = i % NBUF
        return pltpu.async_copy(rows_v.at[b], out_hbm.at[base + i], wsem.at[b])

    gathers = {}
    wbs = {}
    for i in range(LOOKAHEAD):
        gathers[i] = start_gather(i)

    for i in range(CPW):
        b = i % NBUF
        g0, g1 = gathers.pop(i)
        g0.wait()
        g1.wait()

        def add_body(r, carry, _b=b):
            for k in range(DIM // 16):
                sl = pl.ds(k * 16, 16)
                rows_v[_b, r, sl] = rows_v[_b, r, sl] + pe_v[r, sl]
            return carry

        lax.fori_loop(0, SEQ, add_body, 0, unroll=2)
        wbs[i] = start_writeback(i)

        j = i + LOOKAHEAD
        if j < CPW:
            if j >= NBUF:
                wbs.pop(j - NBUF).wait()
            gathers[j] = start_gather(j)

    for i in sorted(wbs):
        wbs[i].wait()


def kernel(X, token_table):
    idx = X.astype(jnp.int32)
    pe = _positional_encoding(SEQ, DIM)

    mesh = plsc.VectorSubcoreMesh(core_axis_name="c", subcore_axis_name="s")
    run = functools.partial(
        pl.kernel,
        mesh=mesh,
        compiler_params=pltpu.CompilerParams(use_tc_tiling_on_sc=False),
        out_type=jax.ShapeDtypeStruct((BATCH, SEQ, DIM), jnp.float32),
        scratch_types=[
            pltpu.VMEM((CPW, SEQ), jnp.int32),
            pltpu.VMEM((NBUF, SEQ, DIM), jnp.float32),
            pltpu.VMEM((SEQ, DIM), jnp.float32),
            pltpu.SemaphoreType.DMA((NBUF,)),
            pltpu.SemaphoreType.DMA((NBUF,)),
        ],
    )(_sc_body)
    return run(idx, token_table, pe)
